# Initial kernel scaffold; baseline (speedup 1.0000x reference)
import jax
import jax.numpy as jnp
from jax.experimental import pallas as pl


def kernel(discrete_expression, normalized_expr, bin_table, W, b):
    raise NotImplementedError("write your pallas kernel here")


# speedup vs baseline: 20.1352x; 20.1352x over previous
import functools

import jax
import jax.numpy as jnp
from jax import lax
from jax.experimental import pallas as pl
from jax.experimental.pallas import tpu as pltpu
from jax.experimental.pallas import tpu_sc as plsc

EMBED = 64
VOCAB = 53
L = 16
DT = 8
IT = 128
REP = VOCAB * EMBED + 1
NHALF = 2


def _make_sc_kernel(b_dim, g_dim, n_workers):
    n_slabs = g_dim * DT * NHALF
    slabs_per_w = n_slabs // n_workers
    swords = DT * b_dim // NHALF
    it_per_slab = b_dim // IT // NHALF
    assert n_slabs % n_workers == 0 and slabs_per_w % 2 == 0

    mesh = plsc.VectorSubcoreMesh(core_axis_name="c", subcore_axis_name="s")

    @functools.partial(
        pl.kernel,
        mesh=mesh,
        out_type=jax.ShapeDtypeStruct((g_dim * EMBED * b_dim,), jnp.float32),
        compiler_params=pltpu.CompilerParams(needs_layout_passes=False),
        scratch_types=[
            pltpu.VMEM((L * REP,), jnp.float32),
            pltpu.VMEM((EMBED,), jnp.float32),
            pltpu.VMEM((EMBED,), jnp.float32),
            pltpu.VMEM((b_dim,), jnp.int32),
            pltpu.VMEM((b_dim,), jnp.float32),
            pltpu.VMEM((swords,), jnp.float32),
            pltpu.VMEM((swords,), jnp.float32),
            pltpu.SemaphoreType.DMA,
            pltpu.SemaphoreType.DMA,
        ],
    )
    def k(idx_hbm, nrm_hbm, tab_hbm, w_hbm, b_hbm, out_hbm,
          tabr_v, w_v, b_v, idx_v, nrm_v, out_v0, out_v1, osem0, osem1):
        wid = lax.axis_index("s") * 2 + lax.axis_index("c")
        s_base = wid * slabs_per_w

        pltpu.sync_copy(w_hbm, w_v)
        pltpu.sync_copy(b_hbm, b_v)
        pltpu.sync_copy(tab_hbm, out_v0.at[pl.ds(0, VOCAB * EMBED)])
        lanes = jax.lax.iota(jnp.int32, L)

        def rep_blk(kb, _):
            src = (out_v0[pl.ds(kb * L, L)]
                   + b_v[pl.ds((kb % (EMBED // L)) * L, L)])
            for r in range(L):
                plsc.store_scatter(tabr_v, [lanes + (r * REP + kb * L)], src)
            return _
        lax.fori_loop(0, VOCAB * EMBED // L, rep_blk, None)

        out_vs = (out_v0, out_v1)
        osems = (osem0, osem1)
        rep_off = lanes * REP

        def do_pair(k2, _):
            for bsel in range(2):
                si = k2 * 2 + bsel
                s = s_base + si
                out_v = out_vs[bsel]

                @pl.when((s % (DT * NHALF) == 0) | (si == 0))
                def _load_cols():
                    col0 = (s // (DT * NHALF)) * b_dim
                    pltpu.sync_copy(idx_hbm.at[pl.ds(col0, b_dim)], idx_v)
                    pltpu.sync_copy(nrm_hbm.at[pl.ds(col0, b_dim)], nrm_v)

                @pl.when(k2 > 0)
                def _reclaim():
                    pltpu.make_async_copy(
                        out_v, out_hbm.at[pl.ds(s * swords, swords)],
                        osems[bsel]).wait()

                dt8 = ((s // NHALF) % DT) * DT
                ih = s % NHALF
                wsp = [plsc.load_gather(
                           w_v, [jnp.full((L,), dt8 + ds, jnp.int32)])
                       for ds in range(DT)]

                @plsc.parallel_loop(0, it_per_slab, unroll=1)
                def do_itile(it):
                    for ilb in range(IT // L):
                        i0 = (ih * it_per_slab + it) * IT + ilb * L
                        ob = it * (DT * IT) + ilb * L
                        idx16 = idx_v[pl.ds(i0, L)]
                        n16 = nrm_v[pl.ds(i0, L)]
                        addr = rep_off + idx16 * EMBED + dt8
                        rows = [plsc.load_gather(tabr_v, [addr + ds])
                                for ds in range(DT)]
                        vals = [rows[ds] + n16 * wsp[ds] for ds in range(DT)]
                        for ds in range(DT):
                            out_v[pl.ds(ob + ds * IT, L)] = vals[ds]

                pltpu.async_copy(out_v, out_hbm.at[pl.ds(s * swords, swords)],
                                 osems[bsel])
            return _
        lax.fori_loop(0, slabs_per_w // 2, do_pair, None)

        for bsel in range(2):
            pltpu.make_async_copy(out_vs[bsel],
                                  out_hbm.at[pl.ds(0, swords)],
                                  osems[bsel]).wait()

    return k


def kernel(discrete_expression, normalized_expr, bin_table, W, b):
    B, G = discrete_expression.shape
    idxT = jnp.swapaxes(discrete_expression, 0, 1).reshape(-1)
    idxT = idxT.astype(jnp.int32)
    nrmT = jnp.swapaxes(normalized_expr, 0, 1).reshape(-1)
    tab = bin_table.reshape(-1)
    w = W.reshape(-1)
    k = _make_sc_kernel(B, G, 32)
    out = k(idxT, nrmT, tab, w, b)
    out5 = out.reshape(G, DT, B // IT, EMBED // DT, IT)
    return out5.transpose(2, 4, 0, 1, 3).reshape(B, G, EMBED)
